# baseline (speedup 1.0000x reference)
import jax
import jax.numpy as jnp
from jax import lax
from jax.experimental import pallas as pl
from jax.experimental.pallas import tpu as pltpu
from jax.experimental.pallas import tpu_sc as plsc

B = 64
H = 224
W = 224
CIN = 5
COUT = 3
LW = W * CIN
LT = 1280
LI = 768
LO = W * COUT
IMG_WORDS = H * LT
TBL_WORDS = B * IMG_WORDS
IDX_ROWS = B * H * LI // 128
OUT_WORDS = B * H * LO

NC = 2
NS = 16
NW = NC * NS
ROWS_PW = B * H // NW
NR = 4
STEPS = ROWS_PW // NR
GATHERS = NR * LI // 128


def _tc_body(x_ref, table_ref, idx_ref):
    v = x_ref[0, :, 0:LW]
    s = (v[0:222, 0:1110] + v[0:222, 10:1120]
         + v[2:224, 0:1110] + v[2:224, 10:1120]) * 0.25
    table_ref[0] = jnp.zeros((H, LT), jnp.float32)
    table_ref[0, 1:223, 5:1115] = s

    row_f = lax.broadcasted_iota(jnp.int32, (H, LW), 0).astype(jnp.float32)
    lane = lax.broadcasted_iota(jnp.int32, (H, LW), 1)
    col_f = (lane // CIN).astype(jnp.float32)
    yi = jnp.clip(jnp.mod(row_f + v, 224.0).astype(jnp.int32), 0, 223)
    xi = jnp.clip(jnp.mod(col_f + v, 224.0).astype(jnp.int32), 0, 223)
    r1 = lax.broadcasted_iota(jnp.int32, (1119, W), 0)
    c1 = lax.broadcasted_iota(jnp.int32, (1119, W), 1)
    sel1 = (r1 == 5 * c1 + 3).astype(jnp.bfloat16)
    xc = jax.lax.dot(xi[:, 0:1119].astype(jnp.bfloat16), sel1,
                     preferred_element_type=jnp.float32)
    yc = jax.lax.dot(yi[:, 1:1120].astype(jnp.bfloat16), sel1,
                     preferred_element_type=jnp.float32)
    r2 = lax.broadcasted_iota(jnp.int32, (W, LO), 0)
    c2 = lax.broadcasted_iota(jnp.int32, (W, LO), 1)
    sel2 = (r2 == c2 // 3).astype(jnp.bfloat16)
    x3 = jax.lax.dot(xc.astype(jnp.bfloat16), sel2,
                     preferred_element_type=jnp.float32)
    y3 = jax.lax.dot(yc.astype(jnp.bfloat16), sel2,
                     preferred_element_type=jnp.float32)
    b_off = pl.program_id(0) * IMG_WORDS
    idx_ref[0, :, 0:LO] = (float(LT) * y3 + 5.0 * x3).astype(jnp.int32) \
        + (c2 % 3) + b_off
    rp = lax.broadcasted_iota(jnp.int32, (H, LI - LO), 0)
    cp = lax.broadcasted_iota(jnp.int32, (H, LI - LO), 1)
    idx_ref[0, :, LO:LI] = rp * LT + cp + b_off


def _tc_stage(x2):
    return pl.pallas_call(
        _tc_body,
        grid=(B,),
        in_specs=[pl.BlockSpec((1, H, LW), lambda b: (b, 0, 0))],
        out_specs=[
            pl.BlockSpec((1, H, LT), lambda b: (b, 0, 0)),
            pl.BlockSpec((1, H, LI), lambda b: (b, 0, 0)),
        ],
        out_shape=[
            jax.ShapeDtypeStruct((B, H, LT), jnp.float32),
            jax.ShapeDtypeStruct((B, H, LI), jnp.int32),
        ],
    )(x2)


def _sc_body(tbl_hbm, idx_hbm, out_hbm, idx_v, rows_v, sem):
    wid = lax.axis_index("s") * NC + lax.axis_index("c")
    row_base = wid * ROWS_PW

    def step(g, carry):
        irow = row_base + g * NR
        i0 = pl.multiple_of(irow * (LI // 128), GATHERS)
        pltpu.sync_copy(idx_hbm.at[pl.ds(i0, GATHERS)], idx_v)
        cps = [pltpu.async_copy(tbl_hbm.at[idx_v.at[j]],
                                rows_v.at[pl.ds(j * 128, 128)], sem)
               for j in range(GATHERS)]
        for cp in cps:
            cp.wait()
        for k in range(NR):
            o0 = pl.multiple_of((irow + k) * LO, 8)
            pltpu.sync_copy(rows_v.at[pl.ds(k * LI, LO)],
                            out_hbm.at[pl.ds(o0, LO)])
        return carry

    lax.fori_loop(0, STEPS, step, 0)


def _sc_stage(tbl, idx):
    mesh = plsc.VectorSubcoreMesh(core_axis_name="c", subcore_axis_name="s")
    fn = pl.kernel(
        _sc_body,
        out_type=jax.ShapeDtypeStruct((OUT_WORDS,), jnp.float32),
        mesh=mesh,
        scratch_types=[
            pltpu.VMEM((GATHERS, 128), jnp.int32),
            pltpu.VMEM((NR * LI,), jnp.float32),
            pltpu.SemaphoreType.DMA,
        ],
    )
    return fn(tbl, idx)


@jax.jit
def kernel(x):
    x2 = x.reshape(B, H, LW)
    table, idx = _tc_stage(x2)
    tbl1 = table.reshape(TBL_WORDS)
    idx2 = idx.reshape(IDX_ROWS, 128)
    out = _sc_stage(tbl1, idx2)
    return out.reshape(B, H, W, COUT)


# speedup vs baseline: 16.0553x; 16.0553x over previous
import jax
import jax.numpy as jnp
from jax import lax
from jax.experimental import pallas as pl
from jax.experimental.pallas import tpu as pltpu
from jax.experimental.pallas import tpu_sc as plsc

B = 64
H = 224
W = 224
CIN = 5
COUT = 3

TROWS_PER_PLANE = 2 * H
TROWS_PER_IMG = COUT * TROWS_PER_PLANE
TBL_ROWS = B * TROWS_PER_IMG
RIGHT_OFF = 28576

NC = 2
NS = 16
NW = NC * NS
IMGS_PW = B // NW
OCH = 32
NCHUNK = H // OCH


def _tc_body(x_ref, tbl_ref, idx_ref):
    v = x_ref[0]
    for c in range(COUT):
        p = v[c]
        s = (p[0:222, 0:222] + p[0:222, 2:224]
             + p[2:224, 0:222] + p[2:224, 2:224]) * 0.25
        r = c * TROWS_PER_PLANE
        tbl_ref[r:r + TROWS_PER_PLANE, :] = jnp.zeros(
            (TROWS_PER_PLANE, 128), jnp.float32)
        tbl_ref[r + 1:r + 223, 1:128] = s[:, 0:127]
        tbl_ref[r + H + 1:r + H + 223, 0:127] = s[:, 95:222]

    ii = lax.broadcasted_iota(jnp.int32, (H, W), 0).astype(jnp.float32)
    jj = lax.broadcasted_iota(jnp.int32, (H, W), 1).astype(jnp.float32)
    yi = jnp.clip(jnp.mod(ii + v[4], 224.0).astype(jnp.int32), 0, 223)
    xi = jnp.clip(jnp.mod(jj + v[3], 224.0).astype(jnp.int32), 0, 223)
    idx_ref[0] = yi * 128 + xi + jnp.where(xi >= 128, RIGHT_OFF, 0)


def _tc_stage(x_p):
    return pl.pallas_call(
        _tc_body,
        grid=(B,),
        in_specs=[pl.BlockSpec((1, CIN, H, W), lambda b: (b, 0, 0, 0))],
        out_specs=[
            pl.BlockSpec((TROWS_PER_IMG, 128), lambda b: (b, 0)),
            pl.BlockSpec((1, H, W), lambda b: (b, 0, 0)),
        ],
        out_shape=[
            jax.ShapeDtypeStruct((TBL_ROWS, 128), jnp.float32),
            jax.ShapeDtypeStruct((B, H, W), jnp.int32),
        ],
    )(x_p)


def _sc_body(tbl_hbm, idx_hbm, out_hbm, tloc, iloc, ob0, ob1, so0, so1):
    wid = lax.axis_index("s") * NC + lax.axis_index("c")
    b0 = wid * IMGS_PW
    obufs = (ob0, ob1)
    sems = (so0, so1)
    pending = [None, None]

    for bb in range(IMGS_PW):
        b = b0 + bb
        pltpu.sync_copy(idx_hbm.at[b], iloc)
        for c in range(COUT):
            r0 = b * TROWS_PER_IMG + c * TROWS_PER_PLANE
            pltpu.sync_copy(
                tbl_hbm.at[pl.ds(r0 * 128, TROWS_PER_PLANE * 128)], tloc)
            for gg in range(NCHUNK):
                pbuf = gg % 2
                if pending[pbuf] is not None:
                    ob_, od_ = pending[pbuf]
                    pltpu.make_async_copy(ob_, od_, sems[pbuf]).wait()
                obuf = obufs[pbuf]

                def fill(rr, carry, _g=gg, _obuf=obuf):
                    row = _g * OCH + rr
                    for t in range(W // 16):
                        pv = iloc[row, pl.ds(t * 16, 16)]
                        _obuf[rr, pl.ds(t * 16, 16)] = plsc.load_gather(
                            tloc, [pv])
                    return carry

                lax.fori_loop(0, OCH, fill, 0)
                dst = out_hbm.at[b, c, pl.ds(gg * OCH, OCH)]
                pltpu.async_copy(obuf, dst, sems[pbuf])
                pending[pbuf] = (obuf, dst)
    for pbuf in range(2):
        if pending[pbuf] is not None:
            ob_, od_ = pending[pbuf]
            pltpu.make_async_copy(ob_, od_, sems[pbuf]).wait()


def _sc_stage(tbl, idx):
    mesh = plsc.VectorSubcoreMesh(core_axis_name="c", subcore_axis_name="s")
    fn = pl.kernel(
        _sc_body,
        out_type=jax.ShapeDtypeStruct((B, COUT, H, W), jnp.float32),
        mesh=mesh,
        compiler_params=pltpu.CompilerParams(needs_layout_passes=False),
        scratch_types=[
            pltpu.VMEM((TROWS_PER_PLANE * 128,), jnp.float32),
            pltpu.VMEM((H, W), jnp.int32),
            pltpu.VMEM((OCH, W), jnp.float32),
            pltpu.VMEM((OCH, W), jnp.float32),
            pltpu.SemaphoreType.DMA,
            pltpu.SemaphoreType.DMA,
        ],
    )
    return fn(tbl, idx)


@jax.jit
def kernel(x):
    x_p = jnp.transpose(x, (0, 3, 1, 2))
    tbl, idx = _tc_stage(x_p)
    out_p = _sc_stage(tbl.reshape(-1), idx)
    return jnp.transpose(out_p, (0, 2, 3, 1))


# speedup vs baseline: 16.5083x; 1.0282x over previous
import jax
import jax.numpy as jnp
from jax import lax
from jax.experimental import pallas as pl
from jax.experimental.pallas import tpu as pltpu
from jax.experimental.pallas import tpu_sc as plsc

B = 64
H = 224
W = 224
CIN = 5
COUT = 3

TROWS_PER_PLANE = 2 * H
TROWS_PER_IMG = COUT * TROWS_PER_PLANE
TBL_ROWS = B * TROWS_PER_IMG
RIGHT_OFF = 28576

NC = 2
NS = 16
NW = NC * NS
BH = B // 2
OCH = 32
NCHUNK = H // OCH


def _tc_body(x_ref, tbl_ref, idx_ref):
    v = x_ref[0]
    for c in range(COUT):
        p = v[c]
        s = (p[0:222, 0:222] + p[0:222, 2:224]
             + p[2:224, 0:222] + p[2:224, 2:224]) * 0.25
        r = c * TROWS_PER_PLANE
        tbl_ref[r:r + TROWS_PER_PLANE, :] = jnp.zeros(
            (TROWS_PER_PLANE, 128), jnp.float32)
        tbl_ref[r + 1:r + 223, 1:128] = s[:, 0:127]
        tbl_ref[r + H + 1:r + H + 223, 0:127] = s[:, 95:222]

    ii = lax.broadcasted_iota(jnp.int32, (H, W), 0).astype(jnp.float32)
    jj = lax.broadcasted_iota(jnp.int32, (H, W), 1).astype(jnp.float32)
    yi = jnp.clip(jnp.mod(ii + v[4], 224.0).astype(jnp.int32), 0, 223)
    xi = jnp.clip(jnp.mod(jj + v[3], 224.0).astype(jnp.int32), 0, 223)
    idx_ref[0] = yi * 128 + xi + jnp.where(xi >= 128, RIGHT_OFF, 0)


def _tc_stage(x_p, base):
    return pl.pallas_call(
        _tc_body,
        grid=(BH,),
        in_specs=[pl.BlockSpec((1, CIN, H, W),
                               lambda b: (b + base, 0, 0, 0))],
        out_specs=[
            pl.BlockSpec((TROWS_PER_IMG, 128), lambda b: (b, 0)),
            pl.BlockSpec((1, H, W), lambda b: (b, 0, 0)),
        ],
        out_shape=[
            jax.ShapeDtypeStruct((BH * TROWS_PER_IMG, 128), jnp.float32),
            jax.ShapeDtypeStruct((BH, H, W), jnp.int32),
        ],
    )(x_p)


def _sc_body(tbl_hbm, idx_hbm, out_hbm, tloc, iloc, ob0, ob1, so0, so1):
    wid = lax.axis_index("s") * NC + lax.axis_index("c")
    obufs = (ob0, ob1)
    sems = (so0, so1)
    pending = [None, None]

    for bb in range(BH // NW):
        b = wid + bb * NW
        pltpu.sync_copy(idx_hbm.at[b], iloc)
        for c in range(COUT):
            r0 = b * TROWS_PER_IMG + c * TROWS_PER_PLANE
            pltpu.sync_copy(
                tbl_hbm.at[pl.ds(r0 * 128, TROWS_PER_PLANE * 128)], tloc)
            for gg in range(NCHUNK):
                pbuf = gg % 2
                if pending[pbuf] is not None:
                    ob_, od_ = pending[pbuf]
                    pltpu.make_async_copy(ob_, od_, sems[pbuf]).wait()
                obuf = obufs[pbuf]

                def fill(rr, carry, _g=gg, _obuf=obuf):
                    row = _g * OCH + rr
                    for t in range(W // 16):
                        pv = iloc[row, pl.ds(t * 16, 16)]
                        _obuf[rr, pl.ds(t * 16, 16)] = plsc.load_gather(
                            tloc, [pv])
                    return carry

                lax.fori_loop(0, OCH, fill, 0)
                dst = out_hbm.at[b, c, pl.ds(gg * OCH, OCH)]
                pltpu.async_copy(obuf, dst, sems[pbuf])
                pending[pbuf] = (obuf, dst)
    for pbuf in range(2):
        if pending[pbuf] is not None:
            ob_, od_ = pending[pbuf]
            pltpu.make_async_copy(ob_, od_, sems[pbuf]).wait()


def _sc_stage(tbl, idx):
    mesh = plsc.VectorSubcoreMesh(core_axis_name="c", subcore_axis_name="s")
    fn = pl.kernel(
        _sc_body,
        out_type=jax.ShapeDtypeStruct((BH, COUT, H, W), jnp.float32),
        mesh=mesh,
        compiler_params=pltpu.CompilerParams(needs_layout_passes=False),
        scratch_types=[
            pltpu.VMEM((TROWS_PER_PLANE * 128,), jnp.float32),
            pltpu.VMEM((H, W), jnp.int32),
            pltpu.VMEM((OCH, W), jnp.float32),
            pltpu.VMEM((OCH, W), jnp.float32),
            pltpu.SemaphoreType.DMA,
            pltpu.SemaphoreType.DMA,
        ],
    )
    return fn(tbl, idx)


@jax.jit
def kernel(x):
    x_p = jnp.transpose(x, (0, 3, 1, 2))
    halves = []
    for base in (0, BH):
        tbl, idx = _tc_stage(x_p, base)
        halves.append(_sc_stage(tbl.reshape(-1), idx))
    out_p = jnp.concatenate(halves, axis=0)
    return jnp.transpose(out_p, (0, 2, 3, 1))
